# baseline (speedup 1.0000x reference)
import functools

import jax
import jax.numpy as jnp
from jax import lax
from jax.experimental import pallas as pl
from jax.experimental.pallas import tpu as pltpu
from jax.experimental.pallas import tpu_sc as plsc

_N = 500000
_C = 100
_NB = 15
_L = 16
_NW = 32
_CW_BIG = 15632
_CW_SMALL = 15616
_NBIG = 18
_CAP = _CW_BIG
_PAD = 15744
_GCH = 128
_NGCH = _PAD // _GCH


def _ece_body(prob_hbm, tp_hbm, out_hbm, tp_v, idx_v, conf_v, acc_c, acc_a,
              stage, sem):
    wid = lax.axis_index("c") * 16 + lax.axis_index("s")
    big = wid < _NBIG
    base = jnp.where(big, wid * _CW_BIG, wid * _CW_SMALL + _NBIG * _L)
    base = pl.multiple_of(base, 16)
    n16 = jnp.where(big, _CW_BIG // 16, _CW_SMALL // 16)

    iota = lax.iota(jnp.int32, _L)
    zeros_i = jnp.zeros((_L,), jnp.int32)
    ones_i = jnp.full((_L,), 1, jnp.int32)
    zeros_f = jnp.zeros((_L,), jnp.float32)

    pltpu.sync_copy(tp_hbm.at[pl.ds(2 * base, 2 * _CW_SMALL)],
                    tp_v.at[pl.ds(0, 2 * _CW_SMALL)])

    @pl.when(big)
    def _():
        pltpu.sync_copy(tp_hbm.at[pl.ds(2 * (base + _CW_SMALL), 2 * _L)],
                        tp_v.at[pl.ds(2 * _CW_SMALL, 2 * _L)])

    for k in range(_L):
        acc_c[pl.ds(k * _L, _L)] = zeros_f
        acc_a[pl.ds(k * _L, _L)] = zeros_f
    for k in range(_CW_SMALL // 16, _PAD // 16):
        idx_v[pl.ds(k * _L, _L)] = zeros_i

    def p1(c, _):
        s = c * _L + iota
        preds = plsc.load_gather(tp_v, [2 * s + 1])
        idx_v[pl.ds(c * _L, _L)] = (base + s) * _C + preds
        return _

    lax.fori_loop(0, n16, p1, None)

    def p2(j, _):
        o = pl.multiple_of(j * _GCH, _GCH)
        pltpu.async_copy(prob_hbm.at[idx_v.at[pl.ds(o, _GCH)]],
                         conf_v.at[pl.ds(o, _GCH)], sem)
        return _

    lax.fori_loop(0, _NGCH, p2, None)
    pltpu.make_async_copy(prob_hbm.at[pl.ds(0, _PAD)], conf_v, sem).wait()

    def p3(c, _):
        s = c * _L + iota
        conf = conf_v[pl.ds(c * _L, _L)]
        preds = plsc.load_gather(tp_v, [2 * s + 1])
        labels = plsc.load_gather(tp_v, [2 * s])
        acc = jnp.where(preds == labels, 1.0, 0.0).astype(jnp.float32)
        t = conf * jnp.float32(_NB)
        ti = t.astype(jnp.int32)
        onedge = t == ti.astype(jnp.float32)
        b = ti - jnp.where(onedge, 1, 0)
        b = jnp.where(conf <= 0.0, _NB, b)
        b = jnp.minimum(jnp.maximum(b, 0), _NB)
        addr = iota * _L + b
        plsc.addupdate_scatter(acc_c, [addr], conf)
        plsc.addupdate_scatter(acc_a, [addr], acc)
        return _

    lax.fori_loop(0, n16, p3, None)

    cs = acc_c[pl.ds(0, _L)]
    as_ = acc_a[pl.ds(0, _L)]
    for l in range(1, _L):
        cs = cs + acc_c[pl.ds(l * _L, _L)]
        as_ = as_ + acc_a[pl.ds(l * _L, _L)]
    stage[pl.ds(0, _L)] = cs
    stage[pl.ds(_L, _L)] = as_
    pltpu.sync_copy(stage, out_hbm.at[wid])


@jax.jit
def _ece_sc(prob_flat, target_pred):
    mesh = plsc.VectorSubcoreMesh(core_axis_name="c", subcore_axis_name="s")
    return pl.kernel(
        _ece_body,
        out_type=jax.ShapeDtypeStruct((_NW, 2 * _L), jnp.float32),
        mesh=mesh,
        compiler_params=pltpu.CompilerParams(needs_layout_passes=False),
        scratch_types=[
            pltpu.VMEM((2 * _CAP,), jnp.int32),
            pltpu.VMEM((_PAD,), jnp.int32),
            pltpu.VMEM((_PAD,), jnp.float32),
            pltpu.VMEM((_L * _L,), jnp.float32),
            pltpu.VMEM((_L * _L,), jnp.float32),
            pltpu.VMEM((2 * _L,), jnp.float32),
            pltpu.SemaphoreType.DMA,
        ],
    )(prob_flat, target_pred)


def kernel(probabilities, target_pred):
    partials = _ece_sc(probabilities.reshape(-1), target_pred.reshape(-1))
    tot = partials.sum(axis=0)
    diff = jnp.abs(tot[_L:_L + _NB] - tot[:_NB])
    return diff.sum() / jnp.float32(_N)


# speedup vs baseline: 1.9740x; 1.9740x over previous
import jax
import jax.numpy as jnp
from jax import lax
from jax.experimental import pallas as pl
from jax.experimental.pallas import tpu as pltpu
from jax.experimental.pallas import tpu_sc as plsc

_N = 500000
_C = 100
_NB = 15
_L = 16
_NW = 32
_CW_BIG = 15632
_CW_SMALL = 15616
_NBIG = 18
_CAP = _CW_BIG
_CH = 256
_NCH = _CW_SMALL // _CH


def _ece_body(prob_hbm, tp_hbm, out_hbm, tp_v, buf_a, buf_b, acc_c, acc_a,
              stage, sem_a, sem_b):
    wid = lax.axis_index("c") * 16 + lax.axis_index("s")
    big = wid < _NBIG
    base = jnp.where(big, wid * _CW_BIG, wid * _CW_SMALL + _NBIG * _L)
    base = pl.multiple_of(base, 16)

    iota = lax.iota(jnp.int32, _L)
    zeros_f = jnp.zeros((_L,), jnp.float32)

    pltpu.sync_copy(tp_hbm.at[pl.ds(2 * base, 2 * _CW_SMALL)],
                    tp_v.at[pl.ds(0, 2 * _CW_SMALL)])

    @pl.when(big)
    def _():
        pltpu.sync_copy(tp_hbm.at[pl.ds(2 * (base + _CW_SMALL), 2 * _L)],
                        tp_v.at[pl.ds(2 * _CW_SMALL, 2 * _L)])

    for k in range(_L):
        acc_c[pl.ds(k * _L, _L)] = zeros_f
        acc_a[pl.ds(k * _L, _L)] = zeros_f

    def start(c, buf, sem, nrows):
        row0 = pl.multiple_of(base + c * _CH, 16)
        return pltpu.async_copy(prob_hbm.at[pl.ds(row0, nrows)],
                                buf.at[pl.ds(0, nrows)], sem)

    def process(c, buf, nq):
        for q in range(nq):
            lr = q * _L + iota
            s = c * _CH + lr
            preds = plsc.load_gather(tp_v, [2 * s + 1])
            labels = plsc.load_gather(tp_v, [2 * s])
            conf = plsc.load_gather(buf, [lr, preds])
            acc = jnp.where(preds == labels, 1.0, 0.0).astype(jnp.float32)
            t = conf * jnp.float32(_NB)
            ti = t.astype(jnp.int32)
            onedge = t == ti.astype(jnp.float32)
            b = ti - jnp.where(onedge, 1, 0)
            b = jnp.where(conf <= 0.0, _NB, b)
            b = jnp.minimum(jnp.maximum(b, 0), _NB)
            addr = iota * _L + b
            plsc.addupdate_scatter(acc_c, [addr], conf)
            plsc.addupdate_scatter(acc_a, [addr], acc)

    start(0, buf_a, sem_a, _CH)

    def pair(p, _):
        ca = 2 * p
        pltpu.make_async_copy(prob_hbm.at[pl.ds(0, _CH)],
                              buf_a.at[pl.ds(0, _CH)], sem_a).wait()
        start(ca + 1, buf_b, sem_b, _CH)
        process(ca, buf_a, _CH // _L)
        pltpu.make_async_copy(prob_hbm.at[pl.ds(0, _CH)],
                              buf_b.at[pl.ds(0, _CH)], sem_b).wait()
        start(ca + 2, buf_a, sem_a, _CH)
        process(ca + 1, buf_b, _CH // _L)
        return _

    lax.fori_loop(0, (_NCH - 1) // 2, pair, None)

    pltpu.make_async_copy(prob_hbm.at[pl.ds(0, _CH)],
                          buf_a.at[pl.ds(0, _CH)], sem_a).wait()
    process(_NCH - 1, buf_a, _CH // _L)

    @pl.when(big)
    def _():
        start(_NCH, buf_b, sem_b, _L)
        pltpu.make_async_copy(prob_hbm.at[pl.ds(0, _L)],
                              buf_b.at[pl.ds(0, _L)], sem_b).wait()
        process(_NCH, buf_b, 1)

    cs = acc_c[pl.ds(0, _L)]
    as_ = acc_a[pl.ds(0, _L)]
    for l in range(1, _L):
        cs = cs + acc_c[pl.ds(l * _L, _L)]
        as_ = as_ + acc_a[pl.ds(l * _L, _L)]
    stage[pl.ds(0, _L)] = cs
    stage[pl.ds(_L, _L)] = as_
    pltpu.sync_copy(stage, out_hbm.at[wid])


@jax.jit
def _ece_sc(prob, tp_flat):
    mesh = plsc.VectorSubcoreMesh(core_axis_name="c", subcore_axis_name="s")
    return pl.kernel(
        _ece_body,
        out_type=jax.ShapeDtypeStruct((_NW, 2 * _L), jnp.float32),
        mesh=mesh,
        compiler_params=pltpu.CompilerParams(needs_layout_passes=False,
                                             use_tc_tiling_on_sc=True),
        scratch_types=[
            pltpu.VMEM((2 * _CAP,), jnp.int32),
            pltpu.VMEM((_CH, _C), jnp.float32),
            pltpu.VMEM((_CH, _C), jnp.float32),
            pltpu.VMEM((_L * _L,), jnp.float32),
            pltpu.VMEM((_L * _L,), jnp.float32),
            pltpu.VMEM((2 * _L,), jnp.float32),
            pltpu.SemaphoreType.DMA,
            pltpu.SemaphoreType.DMA,
        ],
    )(prob, tp_flat)


def kernel(probabilities, target_pred):
    partials = _ece_sc(probabilities, target_pred.reshape(-1))
    tot = partials.sum(axis=0)
    diff = jnp.abs(tot[_L:_L + _NB] - tot[:_NB])
    return diff.sum() / jnp.float32(_N)


# speedup vs baseline: 6.4156x; 3.2500x over previous
import jax
import jax.numpy as jnp
from jax import lax
from jax.experimental import pallas as pl
from jax.experimental.pallas import tpu as pltpu
from jax.experimental.pallas import tpu_sc as plsc

_N = 500000
_C = 100
_NB = 15
_L = 16
_NW = 32
_CH = 128
_CW_BIG = 123 * _CH
_CW_STD = 122 * _CH
_TAIL = 32
_CAP = _CW_BIG


def _ece_body(prob_hbm, lab_hbm, prd_hbm, out_hbm, lab_v, prd_v, buf_a,
              buf_b, acc_c, acc_a, stage, sem_a, sem_b):
    wid = lax.axis_index("c") * 16 + lax.axis_index("s")
    big = wid < 2
    last = wid == _NW - 1
    base = jnp.where(big, wid * _CW_BIG, wid * _CW_STD + 2 * _CH)
    base = pl.multiple_of(base, _CH)
    nch = jnp.where(big | last, 123, 122)

    iota = lax.iota(jnp.int32, _L)
    zeros_f = jnp.zeros((_L,), jnp.float32)

    pltpu.sync_copy(lab_hbm.at[pl.ds(base, _CW_STD)],
                    lab_v.at[pl.ds(0, _CW_STD)])
    pltpu.sync_copy(prd_hbm.at[pl.ds(base, _CW_STD)],
                    prd_v.at[pl.ds(0, _CW_STD)])

    @pl.when(big)
    def _():
        pltpu.sync_copy(lab_hbm.at[pl.ds(base + _CW_STD, _CH)],
                        lab_v.at[pl.ds(_CW_STD, _CH)])
        pltpu.sync_copy(prd_hbm.at[pl.ds(base + _CW_STD, _CH)],
                        prd_v.at[pl.ds(_CW_STD, _CH)])

    @pl.when(last)
    def _():
        pltpu.sync_copy(lab_hbm.at[pl.ds(base + _CW_STD, _TAIL)],
                        lab_v.at[pl.ds(_CW_STD, _TAIL)])
        pltpu.sync_copy(prd_hbm.at[pl.ds(base + _CW_STD, _TAIL)],
                        prd_v.at[pl.ds(_CW_STD, _TAIL)])

    for k in range(_L):
        acc_c[pl.ds(k * _L, _L)] = zeros_f
        acc_a[pl.ds(k * _L, _L)] = zeros_f

    def start(c, buf, sem, ncols):
        c = jnp.minimum(c, nch - 1)
        col0 = pl.multiple_of(base + c * _CH, _CH)
        return pltpu.async_copy(prob_hbm.at[:, pl.ds(col0, ncols)],
                                buf, sem)

    def wait(buf, sem, ncols):
        pltpu.make_async_copy(prob_hbm.at[:, pl.ds(0, ncols)], buf, sem).wait()

    def process(c, buf, ncols, nq):
        for q in range(nq):
            lc = q * _L + iota
            off = c * _CH + q * _L
            preds = prd_v[pl.ds(off, _L)]
            labels = lab_v[pl.ds(off, _L)]
            conf = plsc.load_gather(buf, [preds, lc])
            acc = jnp.where(preds == labels, 1.0, 0.0).astype(jnp.float32)
            t = conf * jnp.float32(_NB)
            ti = t.astype(jnp.int32)
            onedge = t == ti.astype(jnp.float32)
            b = ti - jnp.where(onedge, 1, 0)
            b = jnp.where(conf <= 0.0, _NB, b)
            b = jnp.minimum(jnp.maximum(b, 0), _NB)
            addr = iota * _L + b
            plsc.addupdate_scatter(acc_c, [addr], conf)
            plsc.addupdate_scatter(acc_a, [addr], acc)

    start(0, buf_a, sem_a, _CH)

    def pair(p, _):
        ca = 2 * p
        wait(buf_a, sem_a, _CH)
        start(ca + 1, buf_b, sem_b, _CH)
        process(ca, buf_a, _CH, _CH // _L)
        wait(buf_b, sem_b, _CH)
        start(ca + 2, buf_a, sem_a, _CH)
        process(ca + 1, buf_b, _CH, _CH // _L)
        return _

    lax.fori_loop(0, 61, pair, None)

    wait(buf_a, sem_a, _CH)

    @pl.when(big)
    def _():
        process(122, buf_a, _CH, _CH // _L)

    @pl.when(last)
    def _():
        process(122, buf_a, _CH, _TAIL // _L)

    cs = acc_c[pl.ds(0, _L)]
    as_ = acc_a[pl.ds(0, _L)]
    for l in range(1, _L):
        cs = cs + acc_c[pl.ds(l * _L, _L)]
        as_ = as_ + acc_a[pl.ds(l * _L, _L)]
    stage[pl.ds(0, _L)] = cs
    stage[pl.ds(_L, _L)] = as_
    pltpu.sync_copy(stage, out_hbm.at[wid])


@jax.jit
def _ece_sc(prob_t, labels, preds):
    mesh = plsc.VectorSubcoreMesh(core_axis_name="c", subcore_axis_name="s")
    return pl.kernel(
        _ece_body,
        out_type=jax.ShapeDtypeStruct((_NW, 2 * _L), jnp.float32),
        mesh=mesh,
        compiler_params=pltpu.CompilerParams(needs_layout_passes=False,
                                             use_tc_tiling_on_sc=True),
        scratch_types=[
            pltpu.VMEM((_CAP,), jnp.int32),
            pltpu.VMEM((_CAP,), jnp.int32),
            pltpu.VMEM((_C, _CH), jnp.float32),
            pltpu.VMEM((_C, _CH), jnp.float32),
            pltpu.VMEM((_L * _L,), jnp.float32),
            pltpu.VMEM((_L * _L,), jnp.float32),
            pltpu.VMEM((2 * _L,), jnp.float32),
            pltpu.SemaphoreType.DMA,
            pltpu.SemaphoreType.DMA,
        ],
    )(prob_t, labels, preds)


def kernel(probabilities, target_pred):
    partials = _ece_sc(probabilities.T, target_pred[:, 0], target_pred[:, 1])
    tot = partials.sum(axis=0)
    diff = jnp.abs(tot[_L:_L + _NB] - tot[:_NB])
    return diff.sum() / jnp.float32(_N)


# speedup vs baseline: 7.9367x; 1.2371x over previous
import jax
import jax.numpy as jnp
from jax import lax
from jax.experimental import pallas as pl
from jax.experimental.pallas import tpu as pltpu
from jax.experimental.pallas import tpu_sc as plsc

_N = 500000
_C = 100
_NB = 15
_L = 16
_NW = 32
_CH = 256
_NCH = 61
_EX = 128
_CW_STD = _NCH * _CH
_CW_BIG = _CW_STD + _EX
_TAIL = 32
_CAP = _CW_BIG


def _ece_body(prob_hbm, lab_hbm, prd_hbm, out_hbm, lab_v, prd_v, buf_a,
              buf_b, ex_v, acc_c, acc_a, stage, sem_a, sem_b):
    wid = lax.axis_index("c") * 16 + lax.axis_index("s")
    big = wid < 2
    last = wid == _NW - 1
    base = jnp.where(big, wid * _CW_BIG, wid * _CW_STD + 2 * _EX)
    base = pl.multiple_of(base, _EX)

    iota = lax.iota(jnp.int32, _L)
    zeros_f = jnp.zeros((_L,), jnp.float32)

    pltpu.sync_copy(lab_hbm.at[pl.ds(base, _CW_STD)],
                    lab_v.at[pl.ds(0, _CW_STD)])
    pltpu.sync_copy(prd_hbm.at[pl.ds(base, _CW_STD)],
                    prd_v.at[pl.ds(0, _CW_STD)])

    @pl.when(big)
    def _():
        pltpu.sync_copy(lab_hbm.at[pl.ds(base + _CW_STD, _EX)],
                        lab_v.at[pl.ds(_CW_STD, _EX)])
        pltpu.sync_copy(prd_hbm.at[pl.ds(base + _CW_STD, _EX)],
                        prd_v.at[pl.ds(_CW_STD, _EX)])

    @pl.when(last)
    def _():
        pltpu.sync_copy(lab_hbm.at[pl.ds(base + _CW_STD, _TAIL)],
                        lab_v.at[pl.ds(_CW_STD, _TAIL)])
        pltpu.sync_copy(prd_hbm.at[pl.ds(base + _CW_STD, _TAIL)],
                        prd_v.at[pl.ds(_CW_STD, _TAIL)])

    for k in range(_L):
        acc_c[pl.ds(k * _L, _L)] = zeros_f
        acc_a[pl.ds(k * _L, _L)] = zeros_f

    def start(c, buf, sem):
        c = jnp.minimum(c, _NCH - 1)
        col0 = pl.multiple_of(base + c * _CH, _EX)
        return pltpu.async_copy(prob_hbm.at[:, pl.ds(col0, _CH)], buf, sem)

    def wait(buf, sem):
        pltpu.make_async_copy(prob_hbm.at[:, pl.ds(0, _CH)], buf, sem).wait()

    def process(off0, buf, nq):
        for q in range(nq):
            lc = q * _L + iota
            off = off0 + q * _L
            preds = prd_v[pl.ds(off, _L)]
            labels = lab_v[pl.ds(off, _L)]
            conf = plsc.load_gather(buf, [preds, lc])
            acc = jnp.where(preds == labels, 1.0, 0.0).astype(jnp.float32)
            t = conf * jnp.float32(_NB)
            ti = t.astype(jnp.int32)
            onedge = t == ti.astype(jnp.float32)
            b = ti - jnp.where(onedge, 1, 0)
            b = jnp.where(conf <= 0.0, _NB, b)
            b = jnp.minimum(jnp.maximum(b, 0), _NB)
            addr = iota * _L + b
            plsc.addupdate_scatter(acc_c, [addr], conf)
            plsc.addupdate_scatter(acc_a, [addr], acc)

    start(0, buf_a, sem_a)

    def pair(p, _):
        ca = 2 * p
        wait(buf_a, sem_a)
        start(ca + 1, buf_b, sem_b)
        process(ca * _CH, buf_a, _CH // _L)
        wait(buf_b, sem_b)
        start(ca + 2, buf_a, sem_a)
        process((ca + 1) * _CH, buf_b, _CH // _L)
        return _

    lax.fori_loop(0, _NCH // 2, pair, None)

    wait(buf_a, sem_a)
    process((_NCH - 1) * _CH, buf_a, _CH // _L)

    @pl.when(big | last)
    def _():
        col0 = pl.multiple_of(base + _CW_STD, _EX)
        pltpu.async_copy(prob_hbm.at[:, pl.ds(col0, _EX)], ex_v,
                         sem_b).wait()

    @pl.when(big)
    def _():
        process(_CW_STD, ex_v, _EX // _L)

    @pl.when(last)
    def _():
        process(_CW_STD, ex_v, _TAIL // _L)

    cs = acc_c[pl.ds(0, _L)]
    as_ = acc_a[pl.ds(0, _L)]
    for l in range(1, _L):
        cs = cs + acc_c[pl.ds(l * _L, _L)]
        as_ = as_ + acc_a[pl.ds(l * _L, _L)]
    stage[pl.ds(0, _L)] = cs
    stage[pl.ds(_L, _L)] = as_
    pltpu.sync_copy(stage, out_hbm.at[wid])


@jax.jit
def _ece_sc(prob_t, labels, preds):
    mesh = plsc.VectorSubcoreMesh(core_axis_name="c", subcore_axis_name="s")
    return pl.kernel(
        _ece_body,
        out_type=jax.ShapeDtypeStruct((_NW, 2 * _L), jnp.float32),
        mesh=mesh,
        compiler_params=pltpu.CompilerParams(needs_layout_passes=False,
                                             use_tc_tiling_on_sc=True),
        scratch_types=[
            pltpu.VMEM((_CAP,), jnp.int32),
            pltpu.VMEM((_CAP,), jnp.int32),
            pltpu.VMEM((_C, _CH), jnp.float32),
            pltpu.VMEM((_C, _CH), jnp.float32),
            pltpu.VMEM((_C, _EX), jnp.float32),
            pltpu.VMEM((_L * _L,), jnp.float32),
            pltpu.VMEM((_L * _L,), jnp.float32),
            pltpu.VMEM((2 * _L,), jnp.float32),
            pltpu.SemaphoreType.DMA,
            pltpu.SemaphoreType.DMA,
        ],
    )(prob_t, labels, preds)


def kernel(probabilities, target_pred):
    partials = _ece_sc(probabilities.T, target_pred[:, 0], target_pred[:, 1])
    tot = partials.sum(axis=0)
    diff = jnp.abs(tot[_L:_L + _NB] - tot[:_NB])
    return diff.sum() / jnp.float32(_N)


# speedup vs baseline: 7.9958x; 1.0074x over previous
import jax
import jax.numpy as jnp
from jax import lax
from jax.experimental import pallas as pl
from jax.experimental.pallas import tpu as pltpu
from jax.experimental.pallas import tpu_sc as plsc

_N = 500000
_C = 100
_NB = 15
_L = 16
_NW = 32
_CH = 256
_NCH = 61
_EX = 128
_CW_STD = _NCH * _CH
_CW_BIG = _CW_STD + _EX
_TAIL = 32
_CAP = _CW_BIG


def _ece_body(prob_hbm, pk_hbm, out_hbm, pk_v, buf_a,
              buf_b, ex_v, acc_c, acc_a, stage, sem_a, sem_b):
    wid = lax.axis_index("c") * 16 + lax.axis_index("s")
    big = wid < 2
    last = wid == _NW - 1
    base = jnp.where(big, wid * _CW_BIG, wid * _CW_STD + 2 * _EX)
    base = pl.multiple_of(base, _EX)

    iota = lax.iota(jnp.int32, _L)
    zeros_f = jnp.zeros((_L,), jnp.float32)

    pltpu.sync_copy(pk_hbm.at[pl.ds(base, _CW_STD)],
                    pk_v.at[pl.ds(0, _CW_STD)])

    @pl.when(big)
    def _():
        pltpu.sync_copy(pk_hbm.at[pl.ds(base + _CW_STD, _EX)],
                        pk_v.at[pl.ds(_CW_STD, _EX)])

    @pl.when(last)
    def _():
        pltpu.sync_copy(pk_hbm.at[pl.ds(base + _CW_STD, _TAIL)],
                        pk_v.at[pl.ds(_CW_STD, _TAIL)])

    for k in range(_L):
        acc_c[pl.ds(k * _L, _L)] = zeros_f
        acc_a[pl.ds(k * _L, _L)] = zeros_f

    def start(c, buf, sem):
        c = jnp.minimum(c, _NCH - 1)
        col0 = pl.multiple_of(base + c * _CH, _EX)
        return pltpu.async_copy(prob_hbm.at[:, pl.ds(col0, _CH)], buf, sem)

    def wait(buf, sem):
        pltpu.make_async_copy(prob_hbm.at[:, pl.ds(0, _CH)], buf, sem).wait()

    def process(off0, buf, nq):
        for q in range(nq):
            lc = q * _L + iota
            off = off0 + q * _L
            pk = pk_v[pl.ds(off, _L)]
            preds = pk & 127
            conf = plsc.load_gather(buf, [preds, lc])
            acc = (pk >> 7).astype(jnp.float32)
            t = conf * jnp.float32(_NB)
            ti = t.astype(jnp.int32)
            onedge = t == ti.astype(jnp.float32)
            b = ti - jnp.where(onedge, 1, 0)
            b = jnp.where(conf <= 0.0, _NB, b)
            b = jnp.minimum(jnp.maximum(b, 0), _NB)
            addr = iota * _L + b
            plsc.addupdate_scatter(acc_c, [addr], conf)
            plsc.addupdate_scatter(acc_a, [addr], acc)

    start(0, buf_a, sem_a)

    def pair(p, _):
        ca = 2 * p
        wait(buf_a, sem_a)
        start(ca + 1, buf_b, sem_b)
        process(ca * _CH, buf_a, _CH // _L)
        wait(buf_b, sem_b)
        start(ca + 2, buf_a, sem_a)
        process((ca + 1) * _CH, buf_b, _CH // _L)
        return _

    lax.fori_loop(0, _NCH // 2, pair, None)

    wait(buf_a, sem_a)
    process((_NCH - 1) * _CH, buf_a, _CH // _L)

    @pl.when(big | last)
    def _():
        col0 = pl.multiple_of(base + _CW_STD, _EX)
        pltpu.async_copy(prob_hbm.at[:, pl.ds(col0, _EX)], ex_v,
                         sem_b).wait()

    @pl.when(big)
    def _():
        process(_CW_STD, ex_v, _EX // _L)

    @pl.when(last)
    def _():
        process(_CW_STD, ex_v, _TAIL // _L)

    cs = acc_c[pl.ds(0, _L)]
    as_ = acc_a[pl.ds(0, _L)]
    for l in range(1, _L):
        cs = cs + acc_c[pl.ds(l * _L, _L)]
        as_ = as_ + acc_a[pl.ds(l * _L, _L)]
    stage[pl.ds(0, _L)] = cs
    stage[pl.ds(_L, _L)] = as_
    pltpu.sync_copy(stage, out_hbm.at[wid])


@jax.jit
def _ece_sc(prob_t, packed):
    mesh = plsc.VectorSubcoreMesh(core_axis_name="c", subcore_axis_name="s")
    return pl.kernel(
        _ece_body,
        out_type=jax.ShapeDtypeStruct((_NW, 2 * _L), jnp.float32),
        mesh=mesh,
        compiler_params=pltpu.CompilerParams(needs_layout_passes=False,
                                             use_tc_tiling_on_sc=True),
        scratch_types=[
            pltpu.VMEM((_CAP,), jnp.int32),
            pltpu.VMEM((_C, _CH), jnp.float32),
            pltpu.VMEM((_C, _CH), jnp.float32),
            pltpu.VMEM((_C, _EX), jnp.float32),
            pltpu.VMEM((_L * _L,), jnp.float32),
            pltpu.VMEM((_L * _L,), jnp.float32),
            pltpu.VMEM((2 * _L,), jnp.float32),
            pltpu.SemaphoreType.DMA,
            pltpu.SemaphoreType.DMA,
        ],
    )(prob_t, packed)


def kernel(probabilities, target_pred):
    labels = target_pred[:, 0]
    preds = target_pred[:, 1]
    packed = jnp.where(labels == preds, 128, 0) | preds
    partials = _ece_sc(probabilities.T, packed)
    tot = partials.sum(axis=0)
    diff = jnp.abs(tot[_L:_L + _NB] - tot[:_NB])
    return diff.sum() / jnp.float32(_N)
